# Initial kernel scaffold; baseline (speedup 1.0000x reference)
import functools

import jax
import jax.numpy as jnp
from jax import lax
from jax.experimental import pallas as pl
from jax.experimental.pallas import tpu as pltpu
from jax.experimental.pallas import tpu_sc as plsc

SCALE = 10.0
NC, NS, LANES = 2, 16, 16
NW = NC * NS

WT = 528
WO = 272


def _tables_body(x_ref, h_ref, wx1a, wx1b, we1a, we1b, ta_ref, tb_ref):
  hb = h_ref[0]
  xb = x_ref[0]
  n = hb.shape[0]
  xpad = jnp.concatenate([xb, jnp.zeros((n, WT - 512 - 3), jnp.float32)], 1)
  ta_ref[0] = jnp.concatenate(
      [jnp.dot(hb, wx1a[...], preferred_element_type=jnp.float32),
       jnp.dot(hb, we1a[...], preferred_element_type=jnp.float32), xpad], 1)
  tb_ref[0] = jnp.concatenate(
      [jnp.dot(hb, wx1b[...], preferred_element_type=jnp.float32),
       jnp.dot(hb, we1b[...], preferred_element_type=jnp.float32), xpad], 1)


def _make_tables(x, h, wx1a, wx1b, we1a, we1b):
  B, N, Dh = h.shape
  wspec = lambda s: pl.BlockSpec(s, lambda b: (0,) * len(s))
  return pl.pallas_call(
      _tables_body,
      grid=(B,),
      in_specs=[
          pl.BlockSpec((1, N, 3), lambda b: (b, 0, 0)),
          pl.BlockSpec((1, N, Dh), lambda b: (b, 0, 0)),
          wspec((Dh, 256)), wspec((Dh, 256)), wspec((Dh, 256)), wspec((Dh, 256)),
      ],
      out_specs=[
          pl.BlockSpec((1, N, WT), lambda b: (b, 0, 0)),
          pl.BlockSpec((1, N, WT), lambda b: (b, 0, 0)),
      ],
      out_shape=[
          jax.ShapeDtypeStruct((B, N, WT), jnp.float32),
          jax.ShapeDtypeStruct((B, N, WT), jnp.float32),
      ],
  )(x, h, wx1a, wx1b, we1a, we1b)


def _sc_gather(ta, tb, idx_i, idx_j):
  B, N, _ = ta.shape
  E = idx_i.shape[1]
  epw = E // NW
  CH = 128
  nch = epw // CH
  mesh = plsc.VectorSubcoreMesh(core_axis_name="c", subcore_axis_name="s")

  @functools.partial(
      pl.kernel,
      mesh=mesh,
      out_type=[jax.ShapeDtypeStruct((B, E, WT), jnp.float32),
                jax.ShapeDtypeStruct((B, E, WT), jnp.float32)],
      scratch_types=[
          pltpu.VMEM((epw,), jnp.int32),
          pltpu.VMEM((CH, WT), jnp.float32),
          pltpu.SemaphoreType.DMA,
      ],
  )
  def k(ta_h, tb_h, ii_h, ij_h, ga_h, gb_h, idx_v, rows_v, sem):
    wid = lax.axis_index("s") * NC + lax.axis_index("c")
    base = wid * epw

    def body(b, carry):
      for tab, idx_hbm, out_hbm in ((ta_h, ii_h, ga_h), (tb_h, ij_h, gb_h)):
        pltpu.sync_copy(idx_hbm.at[b, pl.ds(base, epw)], idx_v)
        for c in range(nch):
          pltpu.async_copy(
              tab.at[b].at[idx_v.at[pl.ds(c * CH, CH)]], rows_v, sem).wait()
          pltpu.sync_copy(rows_v, out_hbm.at[b, pl.ds(base + c * CH, CH)])
      return carry

    lax.fori_loop(0, B, body, 0)

  return k(ta, tb, idx_i, idx_j)


def _silu(v):
  return v * jax.nn.sigmoid(v)


def _edge_body(ga_ref, gb_ref, ea_ref, em_ref,
               wx1ea, wx1d, bx1, wx2, bx2, wx3t,
               we1ea, we1d, be1, we2, be2, wat, ba, out_ref):
  ga = ga_ref[0]
  gb = gb_ref[0]
  ea = ea_ref[0]
  msk = em_ref[0]
  eb = ga.shape[0]

  diff = (ga[:, 512:515] - gb[:, 512:515]) * msk
  d2 = jnp.sum(diff * diff, axis=-1, keepdims=True)
  d = jnp.sqrt(d2)

  ea_x = jnp.dot(ea, wx1ea[...], preferred_element_type=jnp.float32)
  t1x = msk * (ga[:, 0:256] + gb[:, 0:256] + ea_x) + (msk * d2) * wx1d[...] + bx1[...]
  u = _silu(t1x)
  w2 = _silu(jnp.dot(u, wx2[...], preferred_element_type=jnp.float32) + bx2[...])
  s = jnp.tanh(jnp.sum(w2 * wx3t[...], axis=-1, keepdims=True))
  xm = diff / (d + 1.0) * s * SCALE

  ea_e = jnp.dot(ea, we1ea[...], preferred_element_type=jnp.float32)
  t1e = msk * (ga[:, 256:512] + gb[:, 256:512] + ea_e) + (msk * d2) * we1d[...] + be1[...]
  m1 = _silu(jnp.dot(_silu(t1e), we2[...], preferred_element_type=jnp.float32) + be2[...])
  att = jax.nn.sigmoid(jnp.sum(m1 * wat[...], axis=-1, keepdims=True) + ba[...])
  emv = att * m1

  out_ref[0] = jnp.concatenate(
      [emv, xm, jnp.zeros((eb, WO - 256 - 3), jnp.float32)], axis=1)


def _edge_mlp(ga, gb, edge_attr, edge_mask, wx1ea, wx1d, bx1, wx2, bx2, wx3t,
              we1ea, we1d, be1, we2, be2, wat, ba):
  B, E, _ = ga.shape
  EBLK = 2048
  wspec = lambda s: pl.BlockSpec(s, lambda b, e: (0,) * len(s))
  return pl.pallas_call(
      _edge_body,
      grid=(B, E // EBLK),
      in_specs=[
          pl.BlockSpec((1, EBLK, WT), lambda b, e: (b, e, 0)),
          pl.BlockSpec((1, EBLK, WT), lambda b, e: (b, e, 0)),
          pl.BlockSpec((1, EBLK, 16), lambda b, e: (b, e, 0)),
          pl.BlockSpec((1, EBLK, 1), lambda b, e: (b, e, 0)),
          wspec((16, 256)), wspec((1, 256)), wspec((1, 256)),
          wspec((256, 256)), wspec((1, 256)), wspec((1, 256)),
          wspec((16, 256)), wspec((1, 256)), wspec((1, 256)),
          wspec((256, 256)), wspec((1, 256)), wspec((1, 1)), wspec((1, 1)),
      ],
      out_specs=pl.BlockSpec((1, EBLK, WO), lambda b, e: (b, e, 0)),
      out_shape=jax.ShapeDtypeStruct((B, E, WO), jnp.float32),
  )(ga, gb, edge_attr, edge_mask, wx1ea, wx1d, bx1, wx2, bx2, wx3t,
    we1ea, we1d, be1, we2, be2, wat, ba)


def _sc_scatter(em_ext, idx4, B, N, E):
  epw = E // NW
  CH = 128
  nch = epw // CH
  rows_per_tile = N // NS
  mesh = plsc.VectorSubcoreMesh(core_axis_name="c", subcore_axis_name="s")
  zeros = jnp.zeros((rows_per_tile, WO), jnp.float32)

  @functools.partial(
      pl.kernel,
      mesh=mesh,
      out_type=jax.ShapeDtypeStruct((NC, B, N, WO), jnp.float32),
      scratch_types=[
          pltpu.VMEM((nch, CH), jnp.int32),
          pltpu.VMEM((CH, WO), jnp.float32),
          pltpu.VMEM_SHARED((B, N, WO), jnp.float32),
      ],
  )
  def k(em_h, idx_h, z_h, out_h, idx_v, rows_v, acc_s):
    cid = lax.axis_index("c")
    sid = lax.axis_index("s")
    wid = sid * NC + cid
    base = wid * epw

    for b in range(B):
      pltpu.sync_copy(z_h, acc_s.at[b, pl.ds(sid * rows_per_tile, rows_per_tile)])
    plsc.subcore_barrier()

    def body(b, carry):
      pltpu.sync_copy(idx_h.at[b, wid], idx_v)
      for c in range(nch):
        pltpu.sync_copy(em_h.at[b, pl.ds(base + c * CH, CH)], rows_v)
        pltpu.sync_copy(rows_v, acc_s.at[b].at[idx_v.at[c]], add=True)
      return carry

    lax.fori_loop(0, B, body, 0)
    plsc.subcore_barrier()

    for b in range(B):
      pltpu.sync_copy(
          acc_s.at[b, pl.ds(sid * rows_per_tile, rows_per_tile)],
          out_h.at[cid, b, pl.ds(sid * rows_per_tile, rows_per_tile)])

  return k(em_ext, idx4, zeros)


def _node_body(x_ref, h_ref, acc_ref, nm_ref, wh1h, wh1e, bh1, wh2, bh2,
               xo_ref, ho_ref):
  xb = x_ref[0]
  hb = h_ref[0]
  nm = nm_ref[0]
  agg = acc_ref[0, 0] + acc_ref[1, 0]
  em_agg = agg[:, 0:256]
  xsum = agg[:, 256:259]
  xo_ref[0] = (xb + xsum) * nm
  t = _silu(jnp.dot(hb, wh1h[...], preferred_element_type=jnp.float32)
            + jnp.dot(em_agg, wh1e[...], preferred_element_type=jnp.float32)
            + bh1[...])
  ho_ref[0] = (hb + jnp.dot(t, wh2[...], preferred_element_type=jnp.float32)
               + bh2[...]) * nm


def _node_update(x, h, acc, node_mask, wh1h, wh1e, bh1, wh2, bh2):
  B, N, Dh = h.shape
  wspec = lambda s: pl.BlockSpec(s, lambda b: (0,) * len(s))
  return pl.pallas_call(
      _node_body,
      grid=(B,),
      in_specs=[
          pl.BlockSpec((1, N, 3), lambda b: (b, 0, 0)),
          pl.BlockSpec((1, N, Dh), lambda b: (b, 0, 0)),
          pl.BlockSpec((NC, 1, N, WO), lambda b: (0, b, 0, 0)),
          pl.BlockSpec((1, N, 1), lambda b: (b, 0, 0)),
          wspec((Dh, 256)), wspec((256, 256)), wspec((1, 256)),
          wspec((256, Dh)), wspec((1, Dh)),
      ],
      out_specs=[
          pl.BlockSpec((1, N, 3), lambda b: (b, 0, 0)),
          pl.BlockSpec((1, N, Dh), lambda b: (b, 0, 0)),
      ],
      out_shape=[
          jax.ShapeDtypeStruct((B, N, 3), jnp.float32),
          jax.ShapeDtypeStruct((B, N, Dh), jnp.float32),
      ],
  )(x, h, acc, node_mask, wh1h, wh1e, bh1, wh2, bh2)


def kernel(x, h, edge_attr, edge_indices, node_mask, edge_mask,
           We1, be1, We2, be2, Wa, ba, Wh1, bh1, Wh2, bh2,
           Wx1, bx1, Wx2, bx2, Wx3):
  B, N, Dh = h.shape
  E = edge_attr.shape[1]

  idx_i = edge_indices[..., 0].astype(jnp.int32)
  idx_j = edge_indices[..., 1].astype(jnp.int32)
  idx4 = idx_i.reshape(B, NW, E // NW // 128, 128)

  ta, tb = _make_tables(x, h, Wx1[0:256], Wx1[256:512], We1[0:256], We1[256:512])
  ga, gb = _sc_gather(ta, tb, idx_i, idx_j)
  em_ext = _edge_mlp(
      ga, gb, edge_attr, edge_mask,
      Wx1[513:529], Wx1[512:513], bx1.reshape(1, 256),
      Wx2, bx2.reshape(1, 256), Wx3.reshape(1, 256),
      We1[513:529], We1[512:513], be1.reshape(1, 256),
      We2, be2.reshape(1, 256), Wa.reshape(1, 256), ba.reshape(1, 1))
  acc = _sc_scatter(em_ext, idx4, B, N, E)
  return _node_update(x, h, acc, node_mask,
                      Wh1[0:Dh], Wh1[Dh:Dh + 256], bh1.reshape(1, 256),
                      Wh2, bh2.reshape(1, Dh))


# speedup vs baseline: 8.8624x; 8.8624x over previous
import functools

import jax
import jax.numpy as jnp
from jax import lax
from jax.experimental import pallas as pl
from jax.experimental.pallas import tpu as pltpu
from jax.experimental.pallas import tpu_sc as plsc

SCALE = 10.0
NC, NS, LANES = 2, 16, 16
NW = NC * NS

WT = 512
WO = 256


def _tables_body(h_ref, wx1a, wx1b, we1a, we1b, ta_ref, tb_ref):
  hb = h_ref[0]
  ta_ref[0] = jnp.concatenate(
      [jnp.dot(hb, wx1a[...], preferred_element_type=jnp.float32),
       jnp.dot(hb, we1a[...], preferred_element_type=jnp.float32)], 1)
  tb_ref[0] = jnp.concatenate(
      [jnp.dot(hb, wx1b[...], preferred_element_type=jnp.float32),
       jnp.dot(hb, we1b[...], preferred_element_type=jnp.float32)], 1)


def _make_tables(h, wx1a, wx1b, we1a, we1b):
  B, N, Dh = h.shape
  wspec = lambda s: pl.BlockSpec(s, lambda b: (0,) * len(s))
  return pl.pallas_call(
      _tables_body,
      grid=(B,),
      in_specs=[
          pl.BlockSpec((1, N, Dh), lambda b: (b, 0, 0)),
          wspec((Dh, 256)), wspec((Dh, 256)), wspec((Dh, 256)), wspec((Dh, 256)),
      ],
      out_specs=[
          pl.BlockSpec((1, N, WT), lambda b: (b, 0, 0)),
          pl.BlockSpec((1, N, WT), lambda b: (b, 0, 0)),
      ],
      out_shape=[
          jax.ShapeDtypeStruct((B, N, WT), jnp.float32),
          jax.ShapeDtypeStruct((B, N, WT), jnp.float32),
      ],
  )(h, wx1a, wx1b, we1a, we1b)


def _sc_gather(ta, tb, idx_i, idx_j):
  B, N, _ = ta.shape
  E = idx_i.shape[1]
  epw = E // NW
  CH = 128
  nch = epw // CH
  mesh = plsc.VectorSubcoreMesh(core_axis_name="c", subcore_axis_name="s")

  @functools.partial(
      pl.kernel,
      mesh=mesh,
      out_type=[jax.ShapeDtypeStruct((B, E, WT), jnp.float32),
                jax.ShapeDtypeStruct((B, E, WT), jnp.float32)],
      scratch_types=[
          pltpu.VMEM((epw,), jnp.int32),
          pltpu.VMEM((CH, WT), jnp.float32),
          pltpu.SemaphoreType.DMA,
      ],
  )
  def k(ta_h, tb_h, ii_h, ij_h, ga_h, gb_h, idx_v, rows_v, sem):
    wid = lax.axis_index("s") * NC + lax.axis_index("c")
    base = wid * epw

    def body(b, carry):
      for tab, idx_hbm, out_hbm in ((ta_h, ii_h, ga_h), (tb_h, ij_h, gb_h)):
        pltpu.sync_copy(idx_hbm.at[b, pl.ds(base, epw)], idx_v)
        for c in range(nch):
          pltpu.async_copy(
              tab.at[b].at[idx_v.at[pl.ds(c * CH, CH)]], rows_v, sem).wait()
          pltpu.sync_copy(rows_v, out_hbm.at[b, pl.ds(base + c * CH, CH)])
      return carry

    lax.fori_loop(0, B, body, 0)

  return k(ta, tb, idx_i, idx_j)


def _silu(v):
  return v * jax.nn.sigmoid(v)


def _edge_body(ga_ref, gb_ref, ea_ref, em_ref, x_ref, ii_ref, ij_ref,
               wx1ea, wx1d, bx1, wx2, bx2, wx3t,
               we1ea, we1d, be1, we2, be2, wat, ba, eacc_ref, xacc_ref):
  ga = ga_ref[0]
  gb = gb_ref[0]
  ea = ea_ref[0]
  msk = em_ref[0]
  xb = x_ref[0]
  ii = ii_ref[0, 0]
  ij = ij_ref[0, 0]
  eb = ga.shape[0]
  n = xb.shape[0]

  lanes = lax.broadcasted_iota(jnp.int32, (eb, n), 1)
  oh_i = (ii[:, None] == lanes).astype(jnp.float32)
  oh_j = (ij[:, None] == lanes).astype(jnp.float32)
  x_i = jnp.dot(oh_i, xb, preferred_element_type=jnp.float32)
  x_j = jnp.dot(oh_j, xb, preferred_element_type=jnp.float32)

  diff = (x_i - x_j) * msk
  d2 = jnp.sum(diff * diff, axis=-1, keepdims=True)
  d = jnp.sqrt(d2)

  ea_x = jnp.dot(ea, wx1ea[...], preferred_element_type=jnp.float32)
  t1x = msk * (ga[:, 0:256] + gb[:, 0:256] + ea_x) + (msk * d2) * wx1d[...] + bx1[...]
  u = _silu(t1x)
  w2 = _silu(jnp.dot(u, wx2[...], preferred_element_type=jnp.float32) + bx2[...])
  s = jnp.tanh(jnp.sum(w2 * wx3t[...], axis=-1, keepdims=True))
  xm = diff / (d + 1.0) * s * SCALE

  ea_e = jnp.dot(ea, we1ea[...], preferred_element_type=jnp.float32)
  t1e = msk * (ga[:, 256:512] + gb[:, 256:512] + ea_e) + (msk * d2) * we1d[...] + be1[...]
  m1 = _silu(jnp.dot(_silu(t1e), we2[...], preferred_element_type=jnp.float32) + be2[...])
  att = jax.nn.sigmoid(jnp.sum(m1 * wat[...], axis=-1, keepdims=True) + ba[...])

  xpart = lax.dot_general(oh_i, xm, (((0,), (0,)), ((), ())),
                          preferred_element_type=jnp.float32)
  epart = lax.dot_general(oh_i, att * m1, (((0,), (0,)), ((), ())),
                          preferred_element_type=jnp.float32)

  @pl.when(pl.program_id(1) == 0)
  def _init():
    xacc_ref[0] = jnp.zeros_like(xacc_ref[0])
    eacc_ref[0] = jnp.zeros_like(eacc_ref[0])

  xacc_ref[0] += xpart
  eacc_ref[0] += epart


def _edge_mlp(ga, gb, edge_attr, edge_mask, x, ii3, ij3,
              wx1ea, wx1d, bx1, wx2, bx2, wx3t,
              we1ea, we1d, be1, we2, be2, wat, ba):
  B, E, _ = ga.shape
  N = x.shape[1]
  EBLK = 2048
  wspec = lambda s: pl.BlockSpec(s, lambda b, e: (0,) * len(s))
  return pl.pallas_call(
      _edge_body,
      grid=(B, E // EBLK),
      in_specs=[
          pl.BlockSpec((1, EBLK, WT), lambda b, e: (b, e, 0)),
          pl.BlockSpec((1, EBLK, WT), lambda b, e: (b, e, 0)),
          pl.BlockSpec((1, EBLK, 16), lambda b, e: (b, e, 0)),
          pl.BlockSpec((1, EBLK, 1), lambda b, e: (b, e, 0)),
          pl.BlockSpec((1, N, 3), lambda b, e: (b, 0, 0)),
          pl.BlockSpec((1, 1, EBLK), lambda b, e: (b * (E // EBLK) + e, 0, 0)),
          pl.BlockSpec((1, 1, EBLK), lambda b, e: (b * (E // EBLK) + e, 0, 0)),
          wspec((16, 256)), wspec((1, 256)), wspec((1, 256)),
          wspec((256, 256)), wspec((1, 256)), wspec((1, 256)),
          wspec((16, 256)), wspec((1, 256)), wspec((1, 256)),
          wspec((256, 256)), wspec((1, 256)), wspec((1, 256)), wspec((1, 1)),
      ],
      out_specs=[
          pl.BlockSpec((1, N, WO), lambda b, e: (b, 0, 0)),
          pl.BlockSpec((1, N, 3), lambda b, e: (b, 0, 0)),
      ],
      out_shape=[
          jax.ShapeDtypeStruct((B, N, WO), jnp.float32),
          jax.ShapeDtypeStruct((B, N, 3), jnp.float32),
      ],
  )(ga, gb, edge_attr, edge_mask, x, ii3, ij3,
    wx1ea, wx1d, bx1, wx2, bx2, wx3t,
    we1ea, we1d, be1, we2, be2, wat, ba)


def _node_body(x_ref, h_ref, acc_ref, xacc_ref, nm_ref, wh1h, wh1e, bh1,
               wh2, bh2, xo_ref, ho_ref):
  xb = x_ref[0]
  hb = h_ref[0]
  nm = nm_ref[0]
  em_agg = acc_ref[0]
  xo_ref[0] = (xb + xacc_ref[0]) * nm
  t = _silu(jnp.dot(hb, wh1h[...], preferred_element_type=jnp.float32)
            + jnp.dot(em_agg, wh1e[...], preferred_element_type=jnp.float32)
            + bh1[...])
  ho_ref[0] = (hb + jnp.dot(t, wh2[...], preferred_element_type=jnp.float32)
               + bh2[...]) * nm


def _node_update(x, h, acc, xacc, node_mask, wh1h, wh1e, bh1, wh2, bh2):
  B, N, Dh = h.shape
  wspec = lambda s: pl.BlockSpec(s, lambda b: (0,) * len(s))
  return pl.pallas_call(
      _node_body,
      grid=(B,),
      in_specs=[
          pl.BlockSpec((1, N, 3), lambda b: (b, 0, 0)),
          pl.BlockSpec((1, N, Dh), lambda b: (b, 0, 0)),
          pl.BlockSpec((1, N, WO), lambda b: (b, 0, 0)),
          pl.BlockSpec((1, N, 3), lambda b: (b, 0, 0)),
          pl.BlockSpec((1, N, 1), lambda b: (b, 0, 0)),
          wspec((Dh, 256)), wspec((256, 256)), wspec((1, 256)),
          wspec((256, Dh)), wspec((1, Dh)),
      ],
      out_specs=[
          pl.BlockSpec((1, N, 3), lambda b: (b, 0, 0)),
          pl.BlockSpec((1, N, Dh), lambda b: (b, 0, 0)),
      ],
      out_shape=[
          jax.ShapeDtypeStruct((B, N, 3), jnp.float32),
          jax.ShapeDtypeStruct((B, N, Dh), jnp.float32),
      ],
  )(x, h, acc, xacc, node_mask, wh1h, wh1e, bh1, wh2, bh2)


def kernel(x, h, edge_attr, edge_indices, node_mask, edge_mask,
           We1, be1, We2, be2, Wa, ba, Wh1, bh1, Wh2, bh2,
           Wx1, bx1, Wx2, bx2, Wx3):
  B, N, Dh = h.shape
  E = edge_attr.shape[1]
  EBLK = 2048

  idx_i = edge_indices[..., 0].astype(jnp.int32)
  idx_j = edge_indices[..., 1].astype(jnp.int32)
  ii3 = idx_i.reshape(B * (E // EBLK), 1, EBLK)
  ij3 = idx_j.reshape(B * (E // EBLK), 1, EBLK)

  ta, tb = _make_tables(h, Wx1[0:256], Wx1[256:512], We1[0:256], We1[256:512])
  ga, gb = _sc_gather(ta, tb, idx_i, idx_j)
  eacc, xacc = _edge_mlp(
      ga, gb, edge_attr, edge_mask, x, ii3, ij3,
      Wx1[513:529], Wx1[512:513], bx1.reshape(1, 256),
      Wx2, bx2.reshape(1, 256), Wx3.reshape(1, 256),
      We1[513:529], We1[512:513], be1.reshape(1, 256),
      We2, be2.reshape(1, 256), Wa.reshape(1, 256), ba.reshape(1, 1))
  return _node_update(x, h, eacc, xacc, node_mask,
                      Wh1[0:Dh], Wh1[Dh:Dh + 256], bh1.reshape(1, 256),
                      Wh2, bh2.reshape(1, Dh))
